# baseline (speedup 1.0000x reference)
import functools

import jax
import jax.numpy as jnp
from jax import lax
from jax.experimental import pallas as pl
from jax.experimental.pallas import tpu as pltpu
from jax.experimental.pallas import tpu_sc as plsc

_B = 16
_D = 2048
_LANES = 16


def _pooler_body(hs_hbm, lens_hbm, out_hbm, lens_v, idx_v, row_v, sem):
    c = lax.axis_index("c")
    s = lax.axis_index("s")

    @pl.when(c == 0)
    def _():
        pltpu.sync_copy(lens_hbm, lens_v)
        lens = lens_v[...]
        last_idx = plsc.cumsum(lens) - 1
        lanes = lax.iota(jnp.int32, _LANES)
        plsc.store_scatter(
            idx_v,
            [jnp.zeros((_LANES,), jnp.int32)],
            last_idx,
            mask=lanes == s,
        )
        pltpu.async_copy(hs_hbm.at[idx_v], row_v, sem).wait()

        acc = jnp.zeros((_LANES,), jnp.float32)
        for j in range(_D // _LANES):
            v = row_v[0, pl.ds(j * _LANES, _LANES)]
            acc = acc + v * v
        ssum = jnp.sum(acc)

        sv = jnp.maximum(jnp.full((_LANES,), ssum, jnp.float32), 1e-24)
        bits = plsc.bitcast(sv, jnp.int32)
        x = plsc.bitcast(jnp.int32(0x5F3759DF) - (bits >> 1), jnp.float32)
        half = sv * 0.5
        for _ in range(3):
            x = x * (1.5 - half * x * x)

        for j in range(_D // _LANES):
            sl = pl.ds(j * _LANES, _LANES)
            row_v[0, sl] = row_v[0, sl] * x

        pltpu.sync_copy(row_v, out_hbm.at[pl.ds(s, 1)])


@jax.jit
def _pooler(hidden_states, prompt_lens):
    mesh = plsc.VectorSubcoreMesh(core_axis_name="c", subcore_axis_name="s")
    return pl.kernel(
        _pooler_body,
        out_type=jax.ShapeDtypeStruct((_B, _D), jnp.float32),
        mesh=mesh,
        compiler_params=pltpu.CompilerParams(needs_layout_passes=False),
        scratch_types=[
            pltpu.VMEM((_B,), jnp.int32),
            pltpu.VMEM((1,), jnp.int32),
            pltpu.VMEM((1, _D), jnp.float32),
            pltpu.SemaphoreType.DMA,
        ],
    )(hidden_states, prompt_lens)


def kernel(hidden_states, prompt_lens):
    return _pooler(hidden_states, prompt_lens.astype(jnp.int32))


# speedup vs baseline: 1.0390x; 1.0390x over previous
import functools

import jax
import jax.numpy as jnp
from jax import lax
from jax.experimental import pallas as pl
from jax.experimental.pallas import tpu as pltpu
from jax.experimental.pallas import tpu_sc as plsc

_B = 16
_D = 2048
_LANES = 16


def _pooler_body(hs_hbm, lens_hbm, out_hbm, lens_v, idx_v, row_v, sem):
    c = lax.axis_index("c")
    s = lax.axis_index("s")

    @pl.when(c == 0)
    def _():
        pltpu.sync_copy(lens_hbm, lens_v)
        lens = lens_v[...]
        last_idx = plsc.cumsum(lens) - 1
        lanes = lax.iota(jnp.int32, _LANES)
        plsc.store_scatter(
            idx_v,
            [jnp.zeros((_LANES,), jnp.int32)],
            last_idx,
            mask=lanes == s,
        )
        pltpu.async_copy(hs_hbm.at[idx_v], row_v, sem).wait()

        def _ssq_step(j, acc):
            base = j * (4 * _LANES)
            for k in range(4):
                v = row_v[0, pl.ds(base + k * _LANES, _LANES)]
                acc = acc + v * v
            return acc

        acc = lax.fori_loop(
            0, _D // (4 * _LANES), _ssq_step, jnp.zeros((_LANES,), jnp.float32)
        )
        ssum = jnp.sum(acc)

        sv = jnp.maximum(jnp.full((_LANES,), ssum, jnp.float32), 1e-24)
        bits = plsc.bitcast(sv, jnp.int32)
        x = plsc.bitcast(jnp.int32(0x5F3759DF) - (bits >> 1), jnp.float32)
        half = sv * 0.5
        for _ in range(3):
            x = x * (1.5 - half * x * x)

        def _scale_step(j, x):
            base = j * (4 * _LANES)
            for k in range(4):
                sl = pl.ds(base + k * _LANES, _LANES)
                row_v[0, sl] = row_v[0, sl] * x
            return x

        lax.fori_loop(0, _D // (4 * _LANES), _scale_step, x)

        pltpu.sync_copy(row_v, out_hbm.at[pl.ds(s, 1)])


@jax.jit
def _pooler(hidden_states, prompt_lens):
    mesh = plsc.VectorSubcoreMesh(core_axis_name="c", subcore_axis_name="s")
    return pl.kernel(
        _pooler_body,
        out_type=jax.ShapeDtypeStruct((_B, _D), jnp.float32),
        mesh=mesh,
        compiler_params=pltpu.CompilerParams(needs_layout_passes=False),
        scratch_types=[
            pltpu.VMEM((_B,), jnp.int32),
            pltpu.VMEM((1,), jnp.int32),
            pltpu.VMEM((1, _D), jnp.float32),
            pltpu.SemaphoreType.DMA,
        ],
    )(hidden_states, prompt_lens)


def kernel(hidden_states, prompt_lens):
    return _pooler(hidden_states, prompt_lens.astype(jnp.int32))


# speedup vs baseline: 1.1187x; 1.0767x over previous
import functools

import jax
import jax.numpy as jnp
from jax import lax
from jax.experimental import pallas as pl
from jax.experimental.pallas import tpu as pltpu
from jax.experimental.pallas import tpu_sc as plsc

_B = 16
_D = 2048
_LANES = 16


def _pooler_body(hs_hbm, lens_hbm, out_hbm, lens_v, idx_v, row_v, sem):
    c = lax.axis_index("c")
    s = lax.axis_index("s")

    @pl.when(c == 0)
    def _():
        pltpu.sync_copy(lens_hbm, lens_v)
        lens = lens_v[...]
        last_idx = plsc.cumsum(lens) - 1
        lanes = lax.iota(jnp.int32, _LANES)
        plsc.store_scatter(
            idx_v,
            [jnp.zeros((_LANES,), jnp.int32)],
            last_idx,
            mask=lanes == s,
        )
        pltpu.async_copy(hs_hbm.at[idx_v], row_v, sem).wait()

        def _ssq_step(j, acc):
            base = j * (4 * _LANES)
            for k in range(4):
                v = row_v[0, pl.ds(base + k * _LANES, _LANES)]
                acc = acc + v * v
            return acc

        acc = lax.fori_loop(
            0, _D // (4 * _LANES), _ssq_step, jnp.zeros((_LANES,), jnp.float32)
        )
        ssum = jnp.sum(acc)

        sv = jnp.maximum(jnp.full((_LANES,), ssum, jnp.float32), 1e-24)
        bits = plsc.bitcast(sv, jnp.int32)
        x = plsc.bitcast(jnp.int32(0x5F3759DF) - (bits >> 1), jnp.float32)
        half = sv * 0.5
        for _ in range(3):
            x = x * (1.5 - half * x * x)

        def _scale_step(j, x):
            base = j * (4 * _LANES)
            for k in range(4):
                sl = pl.ds(base + k * _LANES, _LANES)
                row_v[0, sl] = row_v[0, sl] * x
            return x

        lax.fori_loop(0, _D // (4 * _LANES), _scale_step, x)

        pltpu.sync_copy(row_v, out_hbm.at[pl.ds(s, 1)])


@jax.jit
def _pooler(hidden_states, prompt_lens):
    mesh = plsc.VectorSubcoreMesh(
        core_axis_name="c", subcore_axis_name="s", num_cores=1
    )
    return pl.kernel(
        _pooler_body,
        out_type=jax.ShapeDtypeStruct((_B, _D), jnp.float32),
        mesh=mesh,
        compiler_params=pltpu.CompilerParams(needs_layout_passes=False),
        scratch_types=[
            pltpu.VMEM((_B,), jnp.int32),
            pltpu.VMEM((1,), jnp.int32),
            pltpu.VMEM((1, _D), jnp.float32),
            pltpu.SemaphoreType.DMA,
        ],
    )(hidden_states, prompt_lens)


def kernel(hidden_states, prompt_lens):
    return _pooler(hidden_states, prompt_lens.astype(jnp.int32))
